# baseline (speedup 1.0000x reference)
import jax
import jax.numpy as jnp
from jax.experimental import pallas as pl
from jax.experimental.pallas import tpu as pltpu

NUM_EMB = 1024
EMB_DIM = 256
BATCH = 9216
TEMP = 0.5
BM = 512

LOG2E = 1.4426950408889634
GBOUND = 16.7


def _vq_block(x_ref, g_ref, t_ref, q_ref, idx_ref, loss_ref,
              esq_ref, taug_ref, ones_ref):
    table = t_ref[...]

    @pl.when(pl.program_id(0) == 0)
    def _():
        esq_ref[...] = jnp.sum(table * table, axis=1)[None, :]
        taug_ref[:, :EMB_DIM] = table
        taug_ref[:, EMB_DIM:] = jnp.ones((NUM_EMB, 8), jnp.float32)
        ones_ref[...] = jnp.ones((NUM_EMB, 8), jnp.float32)
        loss_ref[...] = jnp.zeros_like(loss_ref)

    x = x_ref[...]
    xsq = jnp.sum(x * x, axis=1, keepdims=True)
    xe = jax.lax.dot_general(
        x, table, (((1,), (1,)), ((), ())),
        preferred_element_type=jnp.float32)
    d = xsq + esq_ref[...] - 2.0 * xe

    dmin = jnp.min(d, axis=1, keepdims=True)
    t = dmin - d

    cols = jax.lax.broadcasted_iota(jnp.int32, t.shape, 1)
    idx = jnp.min(jnp.where(t == 0.0, cols, NUM_EMB), axis=1)
    idx_ref[...] = idx.astype(jnp.int32)[None, None, :]

    e1 = jnp.exp2(t * LOG2E)
    e1t = e1 * t
    ones = ones_ref[...]
    s1 = jax.lax.dot_general(
        e1, ones, (((1,), (0,)), ((), ())),
        preferred_element_type=jnp.float32)[:, 0:1]
    s2 = jax.lax.dot_general(
        e1t, ones, (((1,), (0,)), ((), ())),
        preferred_element_type=jnp.float32)[:, 0:1]
    kl_rows = jnp.log(float(NUM_EMB)) - jnp.log(s1) + s2 / s1
    loss_ref[...] += jnp.sum(kl_rows).reshape(1, 1)

    ez = jnp.exp2((g_ref[...] + t) * (2.0 * LOG2E) - (2.0 * LOG2E) * GBOUND)
    qaug = jax.lax.dot_general(
        ez, taug_ref[...], (((1,), (0,)), ((), ())),
        preferred_element_type=jnp.float32)
    sz = qaug[:, EMB_DIM:EMB_DIM + 1]
    q_ref[...] = qaug[:, :EMB_DIM] * (1.0 / sz)


@jax.jit
def kernel(x, var, table, gumbel):
    del var
    nb = BATCH // BM
    q, idx3, loss = pl.pallas_call(
        _vq_block,
        grid=(nb,),
        in_specs=[
            pl.BlockSpec((BM, EMB_DIM), lambda i: (i, 0)),
            pl.BlockSpec((BM, NUM_EMB), lambda i: (i, 0)),
            pl.BlockSpec((NUM_EMB, EMB_DIM), lambda i: (0, 0)),
        ],
        out_specs=[
            pl.BlockSpec((BM, EMB_DIM), lambda i: (i, 0)),
            pl.BlockSpec((1, 1, BM), lambda i: (i, 0, 0)),
            pl.BlockSpec((1, 1), lambda i: (0, 0)),
        ],
        out_shape=[
            jax.ShapeDtypeStruct((BATCH, EMB_DIM), jnp.float32),
            jax.ShapeDtypeStruct((nb, 1, BM), jnp.int32),
            jax.ShapeDtypeStruct((1, 1), jnp.float32),
        ],
        scratch_shapes=[
            pltpu.VMEM((1, NUM_EMB), jnp.float32),
            pltpu.VMEM((NUM_EMB, EMB_DIM + 8), jnp.float32),
            pltpu.VMEM((NUM_EMB, 8), jnp.float32),
        ],
    )(x, gumbel, table)
    return q, loss[0, 0] / BATCH, idx3.reshape(BATCH)


# speedup vs baseline: 1.1583x; 1.1583x over previous
import jax
import jax.numpy as jnp
from jax.experimental import pallas as pl
from jax.experimental.pallas import tpu as pltpu

NUM_EMB = 1024
EMB_DIM = 256
BATCH = 9216
TEMP = 0.5
BM = 1024

LOG2E = 1.4426950408889634
GBOUND = 16.7


def _vq_block(x_ref, g_ref, t_ref, q_ref, idx_ref, loss_ref,
              esq_ref, taug_ref, ones_ref):
    table = t_ref[...]

    @pl.when(pl.program_id(0) == 0)
    def _():
        esq_ref[...] = jnp.sum(table * table, axis=1)[None, :]
        taug_ref[:, :EMB_DIM] = table
        taug_ref[:, EMB_DIM:] = jnp.ones((NUM_EMB, 8), jnp.float32)
        ones_ref[...] = jnp.ones((NUM_EMB, 8), jnp.float32)
        loss_ref[...] = jnp.zeros_like(loss_ref)

    x = x_ref[...]
    xsq = jnp.sum(x * x, axis=1, keepdims=True)
    xe = jax.lax.dot_general(
        x, table, (((1,), (1,)), ((), ())),
        preferred_element_type=jnp.float32)
    d = xsq + esq_ref[...] - 2.0 * xe

    dmin = jnp.min(d, axis=1, keepdims=True)
    t = dmin - d

    cols = jax.lax.broadcasted_iota(jnp.int32, t.shape, 1)
    idx = jnp.min(jnp.where(t == 0.0, cols, NUM_EMB), axis=1)
    idx_ref[...] = idx.astype(jnp.int32)[None, None, :]

    e1 = jnp.exp2(t * LOG2E)
    e1t = e1 * t
    ones = ones_ref[...]
    s1 = jax.lax.dot_general(
        e1, ones, (((1,), (0,)), ((), ())),
        preferred_element_type=jnp.float32)[:, 0:1]
    s2 = jax.lax.dot_general(
        e1t, ones, (((1,), (0,)), ((), ())),
        preferred_element_type=jnp.float32)[:, 0:1]
    kl_rows = jnp.log(float(NUM_EMB)) - jnp.log(s1) + s2 / s1
    loss_ref[...] += jnp.sum(kl_rows).reshape(1, 1)

    ez = jnp.exp2((g_ref[...] + t) * (2.0 * LOG2E) - (2.0 * LOG2E) * GBOUND)
    qaug = jax.lax.dot_general(
        ez, taug_ref[...], (((1,), (0,)), ((), ())),
        preferred_element_type=jnp.float32)
    sz = qaug[:, EMB_DIM:EMB_DIM + 1]
    q_ref[...] = qaug[:, :EMB_DIM] * (1.0 / sz)


@jax.jit
def kernel(x, var, table, gumbel):
    del var
    nb = BATCH // BM
    q, idx3, loss = pl.pallas_call(
        _vq_block,
        grid=(nb,),
        in_specs=[
            pl.BlockSpec((BM, EMB_DIM), lambda i: (i, 0)),
            pl.BlockSpec((BM, NUM_EMB), lambda i: (i, 0)),
            pl.BlockSpec((NUM_EMB, EMB_DIM), lambda i: (0, 0)),
        ],
        out_specs=[
            pl.BlockSpec((BM, EMB_DIM), lambda i: (i, 0)),
            pl.BlockSpec((1, 1, BM), lambda i: (i, 0, 0)),
            pl.BlockSpec((1, 1), lambda i: (0, 0)),
        ],
        out_shape=[
            jax.ShapeDtypeStruct((BATCH, EMB_DIM), jnp.float32),
            jax.ShapeDtypeStruct((nb, 1, BM), jnp.int32),
            jax.ShapeDtypeStruct((1, 1), jnp.float32),
        ],
        scratch_shapes=[
            pltpu.VMEM((1, NUM_EMB), jnp.float32),
            pltpu.VMEM((NUM_EMB, EMB_DIM + 8), jnp.float32),
            pltpu.VMEM((NUM_EMB, 8), jnp.float32),
        ],
    )(x, gumbel, table)
    return q, loss[0, 0] / BATCH, idx3.reshape(BATCH)
